# Initial kernel scaffold; baseline (speedup 1.0000x reference)
import jax
import jax.numpy as jnp
from jax.experimental import pallas as pl


def kernel(x, edge_index, edge_attr, agent_neighbor_idx, params):
    raise NotImplementedError("write your pallas kernel here")


# speedup vs baseline: 7.6350x; 7.6350x over previous
import functools

import jax
import jax.numpy as jnp
from jax import lax
from jax.experimental import pallas as pl
from jax.experimental.pallas import tpu as pltpu
from jax.experimental.pallas import tpu_sc as plsc

N_PAD = 10240
K = 128
NW = 32
R = 512
H = 128


def _make_sc_scatter(n_pad, e, d, with_second):
  nch = e // K
  trips = pl.cdiv(nch, NW)
  rows_per = n_pad // 16

  mesh = plsc.VectorSubcoreMesh(core_axis_name="c", subcore_axis_name="s",
                                num_cores=2, num_subcores=16)
  scratch = [
      pltpu.VMEM((K,), jnp.int32),
      pltpu.VMEM((K,), jnp.int32),
      pltpu.VMEM((K, d), jnp.float32),
  ]
  if with_second:
    scratch.append(pltpu.VMEM((K, d), jnp.float32))
  scratch += [
      pltpu.VMEM_SHARED((n_pad, d), jnp.float32),
      pltpu.SemaphoreType.DMA,
  ]

  def body(*refs):
    if with_second:
      (tab_hbm, sec_hbm, src_hbm, dst_hbm, z_hbm, out_hbm,
       srcv, dstv, rows, rows2, acc_sh, sem) = refs
    else:
      (tab_hbm, src_hbm, dst_hbm, z_hbm, out_hbm,
       srcv, dstv, rows, acc_sh, sem) = refs
    cid = lax.axis_index("c")
    sid = lax.axis_index("s")
    wid = sid * 2 + cid
    my_rows = pl.ds(sid * rows_per, rows_per)
    pltpu.sync_copy(z_hbm.at[my_rows], acc_sh.at[my_rows])
    plsc.subcore_barrier()

    @pl.loop(0, trips)
    def _(i):
      c = wid + i * NW

      @pl.when(c < nch)
      def _():
        base = c * K
        pltpu.sync_copy(src_hbm.at[pl.ds(base, K)], srcv)
        pltpu.sync_copy(dst_hbm.at[pl.ds(base, K)], dstv)
        pltpu.async_copy(tab_hbm.at[srcv], rows, sem).wait()
        pltpu.sync_copy(rows, acc_sh.at[dstv], add=True)
        if with_second:
          pltpu.sync_copy(sec_hbm.at[pl.ds(base, K)], rows2)
          pltpu.sync_copy(rows2, acc_sh.at[dstv], add=True)

    plsc.subcore_barrier()
    pltpu.sync_copy(acc_sh.at[my_rows], out_hbm.at[cid, my_rows])

  return pl.kernel(
      body,
      out_type=jax.ShapeDtypeStruct((2, n_pad, d), jnp.float32),
      mesh=mesh,
      scratch_types=scratch,
      compiler_params=pltpu.CompilerParams(use_tc_tiling_on_sc=False),
  )


def _layer_tail(t, g, beta):
  t = jnp.maximum(t, 0.0) * g + beta
  nrm = jnp.sqrt(jnp.sum(t * t, axis=1, keepdims=True))
  return t / jnp.maximum(nrm, 1e-12)


def _tc1_body(xp_ref, acc_ref, winp_ref, m0_ref, wh0_ref, wself0_ref,
              b_in_ref, g0_ref, beta0_ref, o0_ref):
  xp = xp_ref[...]
  h0 = jnp.dot(xp, winp_ref[...], preferred_element_type=jnp.float32)
  h0 = h0 + b_in_ref[...]
  acc = acc_ref[0] + acc_ref[1]
  pre0 = (jnp.dot(acc, m0_ref[...], preferred_element_type=jnp.float32)
          + jnp.dot(h0, wh0_ref[...], preferred_element_type=jnp.float32))
  cnt = acc[:, 6:7] + 1.0
  t = pre0 / cnt + jnp.dot(h0, wself0_ref[...],
                           preferred_element_type=jnp.float32)
  o0_ref[...] = _layer_tail(t, g0_ref[...], beta0_ref[...])


def _tc2_body(o0_ref, s1_ref, acc_ref, idx_ref, wh1_ref, m1_ref, wself1_ref,
              g1_ref, beta1_ref, wja_ref, wjb_ref, bj_ref, ws1_ref, bs1_ref,
              ws2r_ref, bs2_ref, wa1_ref, ba1_ref, wa2_ref, ba2_ref,
              out_ref, nb_ref):
  i = pl.program_id(0)
  nblk = pl.num_programs(0)
  o0 = o0_ref[...]
  ssum = s1_ref[0] + s1_ref[1] + o0
  acc = acc_ref[0] + acc_ref[1]
  pre1 = (jnp.dot(ssum, wh1_ref[...], preferred_element_type=jnp.float32)
          + jnp.dot(acc, m1_ref[...], preferred_element_type=jnp.float32))
  cnt = acc[:, 6:7] + 1.0
  t = pre1 / cnt + jnp.dot(o0, wself1_ref[...],
                           preferred_element_type=jnp.float32)
  o1 = _layer_tail(t, g1_ref[...], beta1_ref[...])

  hj = jnp.dot(o0, wja_ref[...], preferred_element_type=jnp.float32)
  hj = hj + jnp.dot(o1, wjb_ref[...], preferred_element_type=jnp.float32)
  hj = jnp.maximum(hj + bj_ref[...], 0.0)
  hs = jnp.maximum(
      jnp.dot(hj, ws1_ref[...], preferred_element_type=jnp.float32)
      + bs1_ref[...], 0.0)
  s = jnp.sum(hs * ws2r_ref[...], axis=1, keepdims=True) + bs2_ref[0, 0]

  n_row = lax.broadcasted_iota(jnp.int32, (1, R), 1) + i * R
  eq = (idx_ref[...] == n_row).astype(jnp.float32)
  nbp = jnp.dot(eq, s, preferred_element_type=jnp.float32)

  @pl.when(i == 0)
  def _():
    nb_ref[...] = jnp.zeros_like(nb_ref)

  nb_ref[...] += nbp

  @pl.when(i == nblk - 1)
  def _():
    nb = nb_ref[...]
    jrow = lax.broadcasted_iota(jnp.int32, (128, 16), 0)
    mcol = lax.broadcasted_iota(jnp.int32, (128, 16), 1)
    cmat = (jrow % 16 == mcol).astype(jnp.float32)
    arow = lax.broadcasted_iota(jnp.int32, (8, 128), 0)
    jcol = lax.broadcasted_iota(jnp.int32, (8, 128), 1)
    rmat = (jcol // 16 == arow).astype(jnp.float32)
    nb816 = jnp.dot(rmat, nb * cmat,
                    preferred_element_type=jnp.float32)
    a1 = jnp.maximum(
        jnp.dot(nb816, wa1_ref[...], preferred_element_type=jnp.float32)
        + ba1_ref[...], 0.0)
    out_ref[...] = (jnp.dot(a1, wa2_ref[...],
                            preferred_element_type=jnp.float32)
                    + ba2_ref[...])


def _full(shape):
  return pl.BlockSpec(shape, lambda i: (0,) * len(shape))


def kernel(x, edge_index, edge_attr, agent_neighbor_idx, params):
  n, df = x.shape
  e = edge_index.shape[1]
  src, dst = edge_index[0], edge_index[1]

  w_in, b_in = params["W_in"], params["b_in"]
  p0, p1 = params["layer0"], params["layer1"]
  wh0, we0 = p0["W_nb"][:H], p0["W_nb"][H:]
  wh1, we1 = p1["W_nb"][:H], p1["W_nb"][H:]
  m0 = (jnp.zeros((16, H), jnp.float32)
        .at[:df].set(w_in @ wh0).at[df:df + 2].set(we0)
        .at[df + 2].set(b_in @ wh0))
  m1 = jnp.zeros((16, H), jnp.float32).at[df:df + 2].set(we1)
  winp = jnp.zeros((16, H), jnp.float32).at[:df].set(w_in)
  bn_scale = 1.0 / jnp.sqrt(jnp.float32(1.0 + 1e-5))
  g0 = (p0["gamma"] * bn_scale)[None, :]
  g1 = (p1["gamma"] * bn_scale)[None, :]
  beta0, beta1 = p0["beta"][None, :], p1["beta"][None, :]
  wj = params["W_j"]
  wja, wjb = wj[:H], wj[H:]
  bj = params["b_j"][None, :]
  ws1, bs1 = params["W_s1"], params["b_s1"][None, :]
  ws2r = params["W_s2"][:, 0][None, :]
  bs2 = params["b_s2"][None, :]
  wa1, ba1 = params["W_a1"], params["b_a1"][None, :]
  wa2, ba2 = params["W_a2"], params["b_a2"][None, :]
  idxf = agent_neighbor_idx.reshape(-1, 1)

  xp = jnp.pad(x, ((0, N_PAD - n), (0, 16 - df)))
  eap = jnp.concatenate(
      [jnp.zeros((e, df), jnp.float32), edge_attr,
       jnp.ones((e, 1), jnp.float32), jnp.zeros((e, 16 - df - 3), jnp.float32)],
      axis=1)
  z16 = jnp.zeros((N_PAD, 16), jnp.float32)
  z128 = jnp.zeros((N_PAD, H), jnp.float32)
  b_in2 = b_in[None, :]

  acc16 = _make_sc_scatter(N_PAD, e, 16, True)(xp, eap, src, dst, z16)

  nblk = N_PAD // R
  o0 = pl.pallas_call(
      _tc1_body,
      grid=(nblk,),
      in_specs=[
          pl.BlockSpec((R, 16), lambda i: (i, 0)),
          pl.BlockSpec((2, R, 16), lambda i: (0, i, 0)),
          _full((16, H)), _full((16, H)), _full((H, H)), _full((H, H)),
          _full((1, H)), _full((1, H)), _full((1, H)),
      ],
      out_specs=pl.BlockSpec((R, H), lambda i: (i, 0)),
      out_shape=jax.ShapeDtypeStruct((N_PAD, H), jnp.float32),
  )(xp, acc16, winp, m0, wh0, p0["W_self"], b_in2, g0, beta0)

  s1 = _make_sc_scatter(N_PAD, e, H, False)(o0, src, dst, z128)

  logits = pl.pallas_call(
      _tc2_body,
      grid=(nblk,),
      in_specs=[
          pl.BlockSpec((R, H), lambda i: (i, 0)),
          pl.BlockSpec((2, R, H), lambda i: (0, i, 0)),
          pl.BlockSpec((2, R, 16), lambda i: (0, i, 0)),
          _full((128, 1)),
          _full((H, H)), _full((16, H)), _full((H, H)),
          _full((1, H)), _full((1, H)),
          _full((H, H)), _full((H, H)), _full((1, H)),
          _full((H, 64)), _full((1, 64)), _full((1, 64)), _full((1, 1)),
          _full((16, 64)), _full((1, 64)), _full((64, 16)), _full((1, 16)),
      ],
      out_specs=pl.BlockSpec((8, 16), lambda i: (0, 0)),
      out_shape=jax.ShapeDtypeStruct((8, 16), jnp.float32),
      scratch_shapes=[pltpu.VMEM((128, 1), jnp.float32)],
  )(o0, s1, acc16, idxf, wh1, m1, p1["W_self"], g1, beta1,
    wja, wjb, bj, ws1, bs1, ws2r, bs2, wa1, ba1, wa2, ba2)

  return logits


# speedup vs baseline: 9.6175x; 1.2597x over previous
import jax
import jax.numpy as jnp
from jax import lax
from jax.experimental import pallas as pl
from jax.experimental.pallas import tpu as pltpu
from jax.experimental.pallas import tpu_sc as plsc

N_PAD = 10112
K = 128
NW = 32
R = 632
H = 128


def _dist(nch):
  base, rem = nch // NW, nch % NW
  return base, rem, base + (1 if rem else 0)


def _mesh():
  return plsc.VectorSubcoreMesh(core_axis_name="c", subcore_axis_name="s",
                                num_cores=2, num_subcores=16)


def _make_sc_scatter_a(n_pad, nch):
  base, rem, trips = _dist(nch)
  rows_per = n_pad // 16
  nb = 4

  scratch = [
      pltpu.VMEM((trips, K), jnp.int32),
      pltpu.VMEM((trips, K), jnp.int32),
  ]
  scratch += [pltpu.VMEM((K, 16), jnp.float32) for _ in range(nb)]
  scratch += [pltpu.VMEM((K, 16), jnp.float32) for _ in range(nb)]
  scratch.append(pltpu.VMEM_SHARED((n_pad, 16), jnp.float32))
  scratch += [pltpu.SemaphoreType.DMA for _ in range(2 * nb)]

  def body(*refs):
    (tab_hbm, src_hbm, dst_hbm, ea_hbm, z_hbm, out_hbm,
     srcbuf, dstbuf, *rest) = refs
    rows = rest[:nb]
    ebuf = rest[nb:2 * nb]
    acc_sh = rest[2 * nb]
    gsem = rest[2 * nb + 1:3 * nb + 1]
    esem = rest[3 * nb + 1:]

    cid = lax.axis_index("c")
    sid = lax.axis_index("s")
    wid = sid * 2 + cid
    my_rows = pl.ds(sid * rows_per, rows_per)
    chunk0 = wid * base + jnp.minimum(wid, rem)
    mytrips = base + (wid < rem).astype(jnp.int32)

    pltpu.sync_copy(src_hbm.at[pl.ds(chunk0, base)],
                    srcbuf.at[pl.ds(0, base)])
    pltpu.sync_copy(dst_hbm.at[pl.ds(chunk0, base)],
                    dstbuf.at[pl.ds(0, base)])
    if rem:
      @pl.when(wid < rem)
      def _():
        pltpu.sync_copy(src_hbm.at[pl.ds(chunk0 + base, 1)],
                        srcbuf.at[pl.ds(base, 1)])
        pltpu.sync_copy(dst_hbm.at[pl.ds(chunk0 + base, 1)],
                        dstbuf.at[pl.ds(base, 1)])
    pltpu.sync_copy(z_hbm.at[my_rows], acc_sh.at[my_rows])
    plsc.subcore_barrier()

    for b in range(nb):
      pltpu.async_copy(tab_hbm.at[srcbuf.at[b]], rows[b], gsem[b])
      pltpu.async_copy(ea_hbm.at[chunk0 + b], ebuf[b], esem[b])

    @pl.loop(0, pl.cdiv(trips, nb))
    def _(g):
      for b in range(nb):
        t = g * nb + b

        @pl.when(t < mytrips)
        def _():
          pltpu.make_async_copy(tab_hbm.at[srcbuf.at[t]], rows[b],
                                gsem[b]).wait()
          pltpu.sync_copy(rows[b], acc_sh.at[dstbuf.at[t]], add=True)
          pltpu.make_async_copy(ea_hbm.at[0], ebuf[b], esem[b]).wait()
          pltpu.sync_copy(ebuf[b], acc_sh.at[dstbuf.at[t]], add=True)
          tn = t + nb

          @pl.when(tn < mytrips)
          def _():
            pltpu.async_copy(tab_hbm.at[srcbuf.at[tn]], rows[b], gsem[b])
            pltpu.async_copy(ea_hbm.at[chunk0 + tn], ebuf[b], esem[b])

    plsc.subcore_barrier()
    pltpu.sync_copy(acc_sh.at[my_rows], out_hbm.at[cid, my_rows])

  return pl.kernel(
      body,
      out_type=jax.ShapeDtypeStruct((2, n_pad, 16), jnp.float32),
      mesh=_mesh(),
      scratch_types=scratch,
      compiler_params=pltpu.CompilerParams(use_tc_tiling_on_sc=False),
  )


def _make_sc_scatter_b(n_pad, nch):
  base, rem, trips = _dist(nch)
  rows_per = n_pad // 16
  ib, gb = 4, 2

  scratch = [
      pltpu.VMEM((ib, K), jnp.int32),
      pltpu.VMEM((ib, K), jnp.int32),
  ]
  scratch += [pltpu.VMEM((K, H), jnp.float32) for _ in range(gb)]
  scratch.append(pltpu.VMEM_SHARED((n_pad, H), jnp.float32))
  scratch += [pltpu.SemaphoreType.DMA for _ in range(ib + gb)]

  def body(*refs):
    (tab_hbm, src_hbm, dst_hbm, z_hbm, out_hbm, srcring, dstring,
     *rest) = refs
    rows = rest[:gb]
    acc_sh = rest[gb]
    isem = rest[gb + 1:gb + 1 + ib]
    gsem = rest[gb + 1 + ib:]

    cid = lax.axis_index("c")
    sid = lax.axis_index("s")
    wid = sid * 2 + cid
    my_rows = pl.ds(sid * rows_per, rows_per)
    chunk0 = wid * base + jnp.minimum(wid, rem)
    mytrips = base + (wid < rem).astype(jnp.int32)

    pltpu.sync_copy(z_hbm.at[my_rows], acc_sh.at[my_rows])
    plsc.subcore_barrier()

    for j in range(ib):
      pltpu.async_copy(src_hbm.at[chunk0 + j], srcring.at[j], isem[j])
      pltpu.async_copy(dst_hbm.at[chunk0 + j], dstring.at[j], isem[j])
    for t in range(gb):
      pltpu.make_async_copy(src_hbm.at[0], srcring.at[t], isem[t]).wait()
      pltpu.make_async_copy(dst_hbm.at[0], dstring.at[t], isem[t]).wait()
      pltpu.async_copy(tab_hbm.at[srcring.at[t]], rows[t], gsem[t])

    @pl.loop(0, pl.cdiv(trips, ib))
    def _(g):
      for j in range(ib):
        t = g * ib + j
        b = j % gb

        @pl.when(t < mytrips)
        def _():
          pltpu.make_async_copy(tab_hbm.at[srcring.at[j]], rows[b],
                                gsem[b]).wait()
          pltpu.sync_copy(rows[b], acc_sh.at[dstring.at[j]], add=True)
          tn4 = t + ib

          @pl.when(tn4 < mytrips)
          def _():
            pltpu.async_copy(src_hbm.at[chunk0 + tn4], srcring.at[j],
                             isem[j])
            pltpu.async_copy(dst_hbm.at[chunk0 + tn4], dstring.at[j],
                             isem[j])
          tn2 = t + gb

          @pl.when(tn2 < mytrips)
          def _():
            jj = (j + gb) % ib
            pltpu.make_async_copy(src_hbm.at[0], srcring.at[jj],
                                  isem[jj]).wait()
            pltpu.make_async_copy(dst_hbm.at[0], dstring.at[jj],
                                  isem[jj]).wait()
            pltpu.async_copy(tab_hbm.at[srcring.at[jj]], rows[b], gsem[b])

    plsc.subcore_barrier()
    pltpu.sync_copy(acc_sh.at[my_rows], out_hbm.at[cid, my_rows])

  return pl.kernel(
      body,
      out_type=jax.ShapeDtypeStruct((2, n_pad, H), jnp.float32),
      mesh=_mesh(),
      scratch_types=scratch,
      compiler_params=pltpu.CompilerParams(use_tc_tiling_on_sc=False),
  )


def _layer_tail(t, g, beta):
  t = jnp.maximum(t, 0.0) * g + beta
  nrm = jnp.sqrt(jnp.sum(t * t, axis=1, keepdims=True))
  return t / jnp.maximum(nrm, 1e-12)


def _tc1_body(xp_ref, acc_ref, winp_ref, m0_ref, wh0_ref,
              wself0_ref, b_in_ref, g0_ref, beta0_ref, o0_ref):
  xp = xp_ref[...]
  h0 = jnp.dot(xp, winp_ref[...], preferred_element_type=jnp.float32)
  h0 = h0 + b_in_ref[...]
  acc = acc_ref[0] + acc_ref[1]
  pre0 = (jnp.dot(acc, m0_ref[...], preferred_element_type=jnp.float32)
          + jnp.dot(h0, wh0_ref[...], preferred_element_type=jnp.float32))
  cnt = acc[:, 6:7] + 1.0
  t = pre0 / cnt + jnp.dot(h0, wself0_ref[...],
                           preferred_element_type=jnp.float32)
  o0_ref[...] = _layer_tail(t, g0_ref[...], beta0_ref[...])


def _tc2_body(o0_ref, s1_ref, acc_ref, idx_ref, wh1_ref, m1_ref,
              wself1_ref, g1_ref, beta1_ref, wja_ref, wjb_ref, bj_ref,
              ws1_ref, bs1_ref, ws2r_ref, bs2_ref, wa1_ref, ba1_ref, wa2_ref,
              ba2_ref, out_ref, nb_ref):
  i = pl.program_id(0)
  nblk = pl.num_programs(0)
  o0 = o0_ref[...]
  ssum = s1_ref[0] + s1_ref[1] + o0
  acc = acc_ref[0] + acc_ref[1]
  pre1 = (jnp.dot(ssum, wh1_ref[...], preferred_element_type=jnp.float32)
          + jnp.dot(acc, m1_ref[...], preferred_element_type=jnp.float32))
  cnt = acc[:, 6:7] + 1.0
  t = pre1 / cnt + jnp.dot(o0, wself1_ref[...],
                           preferred_element_type=jnp.float32)
  o1 = _layer_tail(t, g1_ref[...], beta1_ref[...])

  hj = jnp.dot(o0, wja_ref[...], preferred_element_type=jnp.float32)
  hj = hj + jnp.dot(o1, wjb_ref[...], preferred_element_type=jnp.float32)
  hj = jnp.maximum(hj + bj_ref[...], 0.0)
  hs = jnp.maximum(
      jnp.dot(hj, ws1_ref[...], preferred_element_type=jnp.float32)
      + bs1_ref[...], 0.0)
  s = jnp.sum(hs * ws2r_ref[...], axis=1, keepdims=True) + bs2_ref[0, 0]

  n_row = lax.broadcasted_iota(jnp.int32, (1, R), 1) + i * R
  eq = (idx_ref[...] == n_row).astype(jnp.float32)
  nbp = jnp.dot(eq, s, preferred_element_type=jnp.float32)

  @pl.when(i == 0)
  def _():
    nb_ref[...] = jnp.zeros_like(nb_ref)

  nb_ref[...] += nbp

  @pl.when(i == nblk - 1)
  def _():
    nb = nb_ref[...]
    jrow = lax.broadcasted_iota(jnp.int32, (128, 16), 0)
    mcol = lax.broadcasted_iota(jnp.int32, (128, 16), 1)
    cmat = (jrow % 16 == mcol).astype(jnp.float32)
    arow = lax.broadcasted_iota(jnp.int32, (8, 128), 0)
    jcol = lax.broadcasted_iota(jnp.int32, (8, 128), 1)
    rmat = (jcol // 16 == arow).astype(jnp.float32)
    nb816 = jnp.dot(rmat, nb * cmat,
                    preferred_element_type=jnp.float32)
    a1 = jnp.maximum(
        jnp.dot(nb816, wa1_ref[...], preferred_element_type=jnp.float32)
        + ba1_ref[...], 0.0)
    out_ref[...] = (jnp.dot(a1, wa2_ref[...],
                            preferred_element_type=jnp.float32)
                    + ba2_ref[...])


def _full(shape):
  return pl.BlockSpec(shape, lambda i: (0,) * len(shape))


def kernel(x, edge_index, edge_attr, agent_neighbor_idx, params):
  n, df = x.shape
  e = edge_index.shape[1]
  assert e % K == 0
  nch = e // K
  src2d = edge_index[0].reshape(nch, K)
  dst2d = edge_index[1].reshape(nch, K)
  df = x.shape[1]
  eap = jnp.pad(edge_attr, ((0, 0), (df, 16 - df - 2))).reshape(nch, K, 16)

  w_in, b_in = params["W_in"], params["b_in"]
  p0, p1 = params["layer0"], params["layer1"]
  wh0, we0 = p0["W_nb"][:H], p0["W_nb"][H:]
  wh1, we1 = p1["W_nb"][:H], p1["W_nb"][H:]
  m0 = (jnp.zeros((16, H), jnp.float32)
        .at[:df].set(w_in @ wh0).at[df:df + 2].set(we0)
        .at[df + 2].set(b_in @ wh0))
  m1 = jnp.zeros((16, H), jnp.float32).at[df:df + 2].set(we1)
  winp = jnp.zeros((16, H), jnp.float32).at[:df].set(w_in)
  bn_scale = 1.0 / jnp.sqrt(jnp.float32(1.0 + 1e-5))
  g0 = (p0["gamma"] * bn_scale)[None, :]
  g1 = (p1["gamma"] * bn_scale)[None, :]
  beta0, beta1 = p0["beta"][None, :], p1["beta"][None, :]
  wj = params["W_j"]
  wja, wjb = wj[:H], wj[H:]
  bj = params["b_j"][None, :]
  ws1, bs1 = params["W_s1"], params["b_s1"][None, :]
  ws2r = params["W_s2"][:, 0][None, :]
  bs2 = params["b_s2"][None, :]
  wa1, ba1 = params["W_a1"], params["b_a1"][None, :]
  wa2, ba2 = params["W_a2"], params["b_a2"][None, :]
  idxf = agent_neighbor_idx.reshape(-1, 1)
  b_in2 = b_in[None, :]

  xp = jnp.pad(x, ((0, N_PAD - n), (0, 16 - df)))
  xp = xp.at[:, df + 2].set(1.0)
  z16 = jnp.zeros((N_PAD, 16), jnp.float32)
  z128 = jnp.zeros((N_PAD, H), jnp.float32)

  acc16 = _make_sc_scatter_a(N_PAD, nch)(xp, src2d, dst2d, eap, z16)

  nblk = N_PAD // R
  o0 = pl.pallas_call(
      _tc1_body,
      grid=(nblk,),
      in_specs=[
          pl.BlockSpec((R, 16), lambda i: (i, 0)),
          pl.BlockSpec((2, R, 16), lambda i: (0, i, 0)),
          _full((16, H)), _full((16, H)), _full((H, H)),
          _full((H, H)), _full((1, H)), _full((1, H)), _full((1, H)),
      ],
      out_specs=pl.BlockSpec((R, H), lambda i: (i, 0)),
      out_shape=jax.ShapeDtypeStruct((N_PAD, H), jnp.float32),
  )(xp, acc16, winp, m0, wh0, p0["W_self"], b_in2, g0, beta0)

  s1 = _make_sc_scatter_b(N_PAD, nch)(o0, src2d, dst2d, z128)

  logits = pl.pallas_call(
      _tc2_body,
      grid=(nblk,),
      in_specs=[
          pl.BlockSpec((R, H), lambda i: (i, 0)),
          pl.BlockSpec((2, R, H), lambda i: (0, i, 0)),
          pl.BlockSpec((2, R, 16), lambda i: (0, i, 0)),
          _full((128, 1)),
          _full((H, H)), _full((16, H)), _full((H, H)),
          _full((1, H)), _full((1, H)),
          _full((H, H)), _full((H, H)), _full((1, H)),
          _full((H, 64)), _full((1, 64)), _full((1, 64)), _full((1, 1)),
          _full((16, 64)), _full((1, 64)), _full((64, 16)), _full((1, 16)),
      ],
      out_specs=pl.BlockSpec((8, 16), lambda i: (0, 0)),
      out_shape=jax.ShapeDtypeStruct((8, 16), jnp.float32),
      scratch_shapes=[pltpu.VMEM((128, 1), jnp.float32)],
  )(o0, s1, acc16, idxf, wh1, m1, p1["W_self"], g1, beta1,
    wja, wjb, bj, ws1, bs1, ws2r, bs2, wa1, ba1, wa2, ba2)

  return logits


# speedup vs baseline: 12.1887x; 1.2673x over previous
import jax
import jax.numpy as jnp
from jax import lax
from jax.experimental import pallas as pl
from jax.experimental.pallas import tpu as pltpu
from jax.experimental.pallas import tpu_sc as plsc

N_PAD = 10112
K = 128
NW = 32
R = 632
H = 128


def _dist(nch):
  base, rem = nch // NW, nch % NW
  return base, rem, base + (1 if rem else 0)


def _mesh():
  return plsc.VectorSubcoreMesh(core_axis_name="c", subcore_axis_name="s",
                                num_cores=2, num_subcores=16)


def _make_sc_scatter_a(n_pad, nch):
  base, rem, trips = _dist(nch)
  rows_per = n_pad // 16
  nb = 4

  scratch = [
      pltpu.VMEM((trips, K), jnp.int32),
      pltpu.VMEM((trips, K), jnp.int32),
  ]
  scratch += [pltpu.VMEM((K, 16), jnp.float32) for _ in range(nb)]
  scratch += [pltpu.VMEM((K, 16), jnp.float32) for _ in range(nb)]
  scratch.append(pltpu.VMEM_SHARED((n_pad, 16), jnp.float32))
  scratch += [pltpu.SemaphoreType.DMA for _ in range(2 * nb)]

  def body(*refs):
    (tab_hbm, src_hbm, dst_hbm, ea_hbm, z_hbm, out_hbm,
     srcbuf, dstbuf, *rest) = refs
    rows = rest[:nb]
    ebuf = rest[nb:2 * nb]
    acc_sh = rest[2 * nb]
    gsem = rest[2 * nb + 1:3 * nb + 1]
    esem = rest[3 * nb + 1:]

    cid = lax.axis_index("c")
    sid = lax.axis_index("s")
    wid = sid * 2 + cid
    my_rows = pl.ds(sid * rows_per, rows_per)
    chunk0 = wid * base + jnp.minimum(wid, rem)
    mytrips = base + (wid < rem).astype(jnp.int32)

    pltpu.sync_copy(src_hbm.at[pl.ds(chunk0, base)],
                    srcbuf.at[pl.ds(0, base)])
    pltpu.sync_copy(dst_hbm.at[pl.ds(chunk0, base)],
                    dstbuf.at[pl.ds(0, base)])
    if rem:
      @pl.when(wid < rem)
      def _():
        pltpu.sync_copy(src_hbm.at[pl.ds(chunk0 + base, 1)],
                        srcbuf.at[pl.ds(base, 1)])
        pltpu.sync_copy(dst_hbm.at[pl.ds(chunk0 + base, 1)],
                        dstbuf.at[pl.ds(base, 1)])
    pltpu.sync_copy(z_hbm.at[my_rows], acc_sh.at[my_rows])
    plsc.subcore_barrier()

    for b in range(nb):
      pltpu.async_copy(tab_hbm.at[srcbuf.at[b]], rows[b], gsem[b])
      pltpu.async_copy(ea_hbm.at[chunk0 + b], ebuf[b], esem[b])

    @pl.loop(0, pl.cdiv(trips, nb))
    def _(g):
      for b in range(nb):
        t = g * nb + b

        @pl.when(t < mytrips)
        def _():
          pltpu.make_async_copy(tab_hbm.at[srcbuf.at[t]], rows[b],
                                gsem[b]).wait()
          pltpu.sync_copy(rows[b], acc_sh.at[dstbuf.at[t]], add=True)
          pltpu.make_async_copy(ea_hbm.at[0], ebuf[b], esem[b]).wait()
          pltpu.sync_copy(ebuf[b], acc_sh.at[dstbuf.at[t]], add=True)
          tn = t + nb

          @pl.when(tn < mytrips)
          def _():
            pltpu.async_copy(tab_hbm.at[srcbuf.at[tn]], rows[b], gsem[b])
            pltpu.async_copy(ea_hbm.at[chunk0 + tn], ebuf[b], esem[b])

    plsc.subcore_barrier()
    pltpu.sync_copy(acc_sh.at[my_rows], out_hbm.at[cid, my_rows])

  return pl.kernel(
      body,
      out_type=jax.ShapeDtypeStruct((2, n_pad, 16), jnp.float32),
      mesh=_mesh(),
      scratch_types=scratch,
      compiler_params=pltpu.CompilerParams(use_tc_tiling_on_sc=False),
  )


def _make_sc_scatter_b(n_pad, nch):
  base, rem, trips = _dist(nch)
  rows_per = n_pad // 16
  ib, gb = 4, 2

  scratch = [
      pltpu.VMEM((ib, K), jnp.int32),
      pltpu.VMEM((ib, K), jnp.int32),
  ]
  scratch += [pltpu.VMEM((K, H), jnp.float32) for _ in range(gb)]
  scratch.append(pltpu.VMEM_SHARED((n_pad, H), jnp.float32))
  scratch += [pltpu.SemaphoreType.DMA for _ in range(ib + gb)]

  def body(*refs):
    (tab_hbm, src_hbm, dst_hbm, z_hbm, out_hbm, srcring, dstring,
     *rest) = refs
    rows = rest[:gb]
    acc_sh = rest[gb]
    isem = rest[gb + 1:gb + 1 + ib]
    gsem = rest[gb + 1 + ib:]

    cid = lax.axis_index("c")
    sid = lax.axis_index("s")
    wid = sid * 2 + cid
    my_rows = pl.ds(sid * rows_per, rows_per)
    chunk0 = wid * base + jnp.minimum(wid, rem)
    mytrips = base + (wid < rem).astype(jnp.int32)

    pltpu.sync_copy(z_hbm.at[my_rows], acc_sh.at[my_rows])
    plsc.subcore_barrier()

    for j in range(ib):
      pltpu.async_copy(src_hbm.at[chunk0 + j], srcring.at[j], isem[j])
      pltpu.async_copy(dst_hbm.at[chunk0 + j], dstring.at[j], isem[j])
    for t in range(gb):
      pltpu.make_async_copy(src_hbm.at[0], srcring.at[t], isem[t]).wait()
      pltpu.make_async_copy(dst_hbm.at[0], dstring.at[t], isem[t]).wait()
      pltpu.async_copy(tab_hbm.at[srcring.at[t]], rows[t], gsem[t])

    @pl.loop(0, pl.cdiv(trips, ib))
    def _(g):
      for j in range(ib):
        t = g * ib + j
        b = j % gb

        @pl.when(t < mytrips)
        def _():
          pltpu.make_async_copy(tab_hbm.at[srcring.at[j]], rows[b],
                                gsem[b]).wait()
          pltpu.sync_copy(rows[b], acc_sh.at[dstring.at[j]], add=True)
          tn4 = t + ib

          @pl.when(tn4 < mytrips)
          def _():
            pltpu.async_copy(src_hbm.at[chunk0 + tn4], srcring.at[j],
                             isem[j])
            pltpu.async_copy(dst_hbm.at[chunk0 + tn4], dstring.at[j],
                             isem[j])
          tn2 = t + gb

          @pl.when(tn2 < mytrips)
          def _():
            jj = (j + gb) % ib
            pltpu.make_async_copy(src_hbm.at[0], srcring.at[jj],
                                  isem[jj]).wait()
            pltpu.make_async_copy(dst_hbm.at[0], dstring.at[jj],
                                  isem[jj]).wait()
            pltpu.async_copy(tab_hbm.at[srcring.at[jj]], rows[b], gsem[b])

    plsc.subcore_barrier()
    pltpu.sync_copy(acc_sh.at[my_rows], out_hbm.at[cid, my_rows])

  return pl.kernel(
      body,
      out_type=jax.ShapeDtypeStruct((2, n_pad, H), jnp.float32),
      mesh=_mesh(),
      scratch_types=scratch,
      compiler_params=pltpu.CompilerParams(use_tc_tiling_on_sc=False),
  )


def _fmt_body(ea16_ref, x32_ref, p_ref, q_ref, ones_ref, eap_ref, xp_ref):
  i = pl.program_id(0)
  eap_ref[...] = jnp.dot(ea16_ref[...], p_ref[...],
                         preferred_element_type=jnp.float32)

  @pl.when(i == 0)
  def _():
    xp_ref[...] = (jnp.dot(x32_ref[...], q_ref[...],
                           preferred_element_type=jnp.float32)
                   + ones_ref[...])


def _layer_tail(t, g, beta):
  t = jnp.maximum(t, 0.0) * g + beta
  nrm = jnp.sqrt(jnp.sum(t * t, axis=1, keepdims=True))
  return t / jnp.maximum(nrm, 1e-12)


def _tc1_body(xp_ref, acc_ref, winp_ref, m0_ref, wh0_ref,
              wself0_ref, b_in_ref, g0_ref, beta0_ref, o0_ref):
  xp = xp_ref[...]
  h0 = jnp.dot(xp, winp_ref[...], preferred_element_type=jnp.float32)
  h0 = h0 + b_in_ref[...]
  acc = acc_ref[0] + acc_ref[1]
  pre0 = (jnp.dot(acc, m0_ref[...], preferred_element_type=jnp.float32)
          + jnp.dot(h0, wh0_ref[...], preferred_element_type=jnp.float32))
  cnt = acc[:, 6:7] + 1.0
  t = pre0 / cnt + jnp.dot(h0, wself0_ref[...],
                           preferred_element_type=jnp.float32)
  o0_ref[...] = _layer_tail(t, g0_ref[...], beta0_ref[...])


def _tc2_body(o0_ref, s1_ref, acc_ref, idx_ref, wh1_ref, m1_ref,
              wself1_ref, g1_ref, beta1_ref, wja_ref, wjb_ref, bj_ref,
              ws1_ref, bs1_ref, ws2r_ref, bs2_ref, wa1_ref, ba1_ref, wa2_ref,
              ba2_ref, out_ref, nb_ref):
  i = pl.program_id(0)
  nblk = pl.num_programs(0)
  o0 = o0_ref[...]
  ssum = s1_ref[0] + s1_ref[1] + o0
  acc = acc_ref[0] + acc_ref[1]
  pre1 = (jnp.dot(ssum, wh1_ref[...], preferred_element_type=jnp.float32)
          + jnp.dot(acc, m1_ref[...], preferred_element_type=jnp.float32))
  cnt = acc[:, 6:7] + 1.0
  t = pre1 / cnt + jnp.dot(o0, wself1_ref[...],
                           preferred_element_type=jnp.float32)
  o1 = _layer_tail(t, g1_ref[...], beta1_ref[...])

  hj = jnp.dot(o0, wja_ref[...], preferred_element_type=jnp.float32)
  hj = hj + jnp.dot(o1, wjb_ref[...], preferred_element_type=jnp.float32)
  hj = jnp.maximum(hj + bj_ref[...], 0.0)
  hs = jnp.maximum(
      jnp.dot(hj, ws1_ref[...], preferred_element_type=jnp.float32)
      + bs1_ref[...], 0.0)
  s = jnp.sum(hs * ws2r_ref[...], axis=1, keepdims=True) + bs2_ref[0, 0]

  n_row = lax.broadcasted_iota(jnp.int32, (1, R), 1) + i * R
  eq = (idx_ref[...] == n_row).astype(jnp.float32)
  nbp = jnp.dot(eq, s, preferred_element_type=jnp.float32)

  @pl.when(i == 0)
  def _():
    nb_ref[...] = jnp.zeros_like(nb_ref)

  nb_ref[...] += nbp

  @pl.when(i == nblk - 1)
  def _():
    nb = nb_ref[...]
    jrow = lax.broadcasted_iota(jnp.int32, (128, 16), 0)
    mcol = lax.broadcasted_iota(jnp.int32, (128, 16), 1)
    cmat = (jrow % 16 == mcol).astype(jnp.float32)
    arow = lax.broadcasted_iota(jnp.int32, (8, 128), 0)
    jcol = lax.broadcasted_iota(jnp.int32, (8, 128), 1)
    rmat = (jcol // 16 == arow).astype(jnp.float32)
    nb816 = jnp.dot(rmat, nb * cmat,
                    preferred_element_type=jnp.float32)
    a1 = jnp.maximum(
        jnp.dot(nb816, wa1_ref[...], preferred_element_type=jnp.float32)
        + ba1_ref[...], 0.0)
    out_ref[...] = (jnp.dot(a1, wa2_ref[...],
                            preferred_element_type=jnp.float32)
                    + ba2_ref[...])


def _full(shape):
  return pl.BlockSpec(shape, lambda i: (0,) * len(shape))


def kernel(x, edge_index, edge_attr, agent_neighbor_idx, params):
  n, df = x.shape
  e = edge_index.shape[1]
  assert e % K == 0
  nch = e // K
  src2d = edge_index[0].reshape(nch, K)
  dst2d = edge_index[1].reshape(nch, K)

  w_in, b_in = params["W_in"], params["b_in"]
  p0, p1 = params["layer0"], params["layer1"]
  wh0, we0 = p0["W_nb"][:H], p0["W_nb"][H:]
  wh1, we1 = p1["W_nb"][:H], p1["W_nb"][H:]
  m0 = (jnp.zeros((16, H), jnp.float32)
        .at[:df].set(w_in @ wh0).at[df:df + 2].set(we0)
        .at[df + 2].set(b_in @ wh0))
  m1 = jnp.zeros((16, H), jnp.float32).at[df:df + 2].set(we1)
  winp = jnp.zeros((16, H), jnp.float32).at[:df].set(w_in)
  bn_scale = 1.0 / jnp.sqrt(jnp.float32(1.0 + 1e-5))
  g0 = (p0["gamma"] * bn_scale)[None, :]
  g1 = (p1["gamma"] * bn_scale)[None, :]
  beta0, beta1 = p0["beta"][None, :], p1["beta"][None, :]
  wj = params["W_j"]
  wja, wjb = wj[:H], wj[H:]
  bj = params["b_j"][None, :]
  ws1, bs1 = params["W_s1"], params["b_s1"][None, :]
  ws2r = params["W_s2"][:, 0][None, :]
  bs2 = params["b_s2"][None, :]
  wa1, ba1 = params["W_a1"], params["b_a1"][None, :]
  wa2, ba2 = params["W_a2"], params["b_a2"][None, :]
  idxf = agent_neighbor_idx.reshape(-1, 1)
  b_in2 = b_in[None, :]

  i16 = jnp.arange(16)
  i32r = jnp.arange(32)
  i128 = jnp.arange(128)
  pmat = (i128[None, :] == (16 * (i16 // 2) + df + (i16 & 1))[:, None]
          ).astype(jnp.float32)
  qmat = (i128[None, :] == (16 * (i32r // df) + (i32r % df))[:, None]
          ).astype(jnp.float32)
  ones_row = ((i128 % 16) == (df + 2)).astype(jnp.float32)[None, :]
  ea16 = edge_attr.reshape(e // 8, 16)
  x32 = jnp.pad(x, ((0, N_PAD - n), (0, 0))).reshape(N_PAD // 8, 8 * df)

  eb = e // 8 // 8
  eap2, xp2 = pl.pallas_call(
      _fmt_body,
      grid=(8,),
      in_specs=[
          pl.BlockSpec((eb, 16), lambda i: (i, 0)),
          _full((N_PAD // 8, 8 * df)),
          _full((16, 128)), _full((8 * df, 128)), _full((1, 128)),
      ],
      out_specs=[
          pl.BlockSpec((eb, 128), lambda i: (i, 0)),
          pl.BlockSpec((N_PAD // 8, 128), lambda i: (0, 0)),
      ],
      out_shape=[jax.ShapeDtypeStruct((e // 8, 128), jnp.float32),
                 jax.ShapeDtypeStruct((N_PAD // 8, 128), jnp.float32)],
  )(ea16, x32, pmat, qmat, ones_row)
  eap = eap2.reshape(nch, K, 16)
  xp = xp2.reshape(N_PAD, 16)
  z16 = jnp.zeros((N_PAD, 16), jnp.float32)
  z128 = jnp.zeros((N_PAD, H), jnp.float32)

  acc16 = _make_sc_scatter_a(N_PAD, nch)(xp, src2d, dst2d, eap, z16)

  nblk = N_PAD // R
  o0 = pl.pallas_call(
      _tc1_body,
      grid=(nblk,),
      in_specs=[
          pl.BlockSpec((R, 16), lambda i: (i, 0)),
          pl.BlockSpec((2, R, 16), lambda i: (0, i, 0)),
          _full((16, H)), _full((16, H)), _full((H, H)),
          _full((H, H)), _full((1, H)), _full((1, H)), _full((1, H)),
      ],
      out_specs=pl.BlockSpec((R, H), lambda i: (i, 0)),
      out_shape=jax.ShapeDtypeStruct((N_PAD, H), jnp.float32),
  )(xp, acc16, winp, m0, wh0, p0["W_self"], b_in2, g0, beta0)

  s1 = _make_sc_scatter_b(N_PAD, nch)(o0, src2d, dst2d, z128)

  logits = pl.pallas_call(
      _tc2_body,
      grid=(nblk,),
      in_specs=[
          pl.BlockSpec((R, H), lambda i: (i, 0)),
          pl.BlockSpec((2, R, H), lambda i: (0, i, 0)),
          pl.BlockSpec((2, R, 16), lambda i: (0, i, 0)),
          _full((128, 1)),
          _full((H, H)), _full((16, H)), _full((H, H)),
          _full((1, H)), _full((1, H)),
          _full((H, H)), _full((H, H)), _full((1, H)),
          _full((H, 64)), _full((1, 64)), _full((1, 64)), _full((1, 1)),
          _full((16, 64)), _full((1, 64)), _full((64, 16)), _full((1, 16)),
      ],
      out_specs=pl.BlockSpec((8, 16), lambda i: (0, 0)),
      out_shape=jax.ShapeDtypeStruct((8, 16), jnp.float32),
      scratch_shapes=[pltpu.VMEM((128, 1), jnp.float32)],
  )(o0, s1, acc16, idxf, wh1, m1, p1["W_self"], g1, beta1,
    wja, wjb, bj, ws1, bs1, ws2r, bs2, wa1, ba1, wa2, ba2)

  return logits
